# baseline (speedup 1.0000x reference)
import jax
import jax.numpy as jnp
from jax.experimental import pallas as pl

NUM_CATEGORIES = 1000


def _onehot_body(inp_ref, out_ref):
    v = inp_ref[...]
    iota = jax.lax.broadcasted_iota(
        jnp.int32, (1, NUM_CATEGORIES, v.shape[2]), 1
    )
    out_ref[...] = (iota == v).astype(jnp.float32)


def kernel(inputs):
    batch, nfeat = inputs.shape
    vt = inputs.astype(jnp.int32).T.reshape(nfeat, 1, batch)
    out_t = pl.pallas_call(
        _onehot_body,
        grid=(nfeat,),
        in_specs=[pl.BlockSpec((1, 1, batch), lambda f: (f, 0, 0))],
        out_specs=pl.BlockSpec((1, NUM_CATEGORIES, batch), lambda f: (f, 0, 0)),
        out_shape=jax.ShapeDtypeStruct((nfeat, NUM_CATEGORIES, batch), jnp.float32),
    )(vt)
    return jnp.transpose(out_t, (2, 0, 1))
